# baseline (speedup 1.0000x reference)
import functools

import jax
import jax.numpy as jnp
from jax import lax
from jax.experimental import pallas as pl
from jax.experimental.pallas import tpu as pltpu
from jax.experimental.pallas import tpu_sc as plsc

N = 10000
E = 320000
D = 128
HIDDEN = 256
OUT_C = 128

L = 16
NC = 2
NS = 16
CHUNK = 128
EPAD = 327680
NCHUNK = EPAD // CHUNK
CPT = NCHUNK // NS
SUP = 8
NSUP = CPT // SUP
NROW = 640
NROW15 = N - 15 * NROW

_INTERP = False
_MESH = plsc.VectorSubcoreMesh(core_axis_name="c", subcore_axis_name="s",
                               num_cores=NC, num_subcores=NS)
_F32 = jnp.float32


def _rsqrt16(d):
    i = lax.bitcast_convert_type(d, jnp.int32)
    i = jnp.int32(0x5F3759DF) - lax.shift_right_logical(i, 1)
    y = lax.bitcast_convert_type(i, _F32)
    half = 0.5 * d
    for _ in range(3):
        y = y * (1.5 - half * y * y)
    return jnp.where(d > 0.5, y, 0.0)


def _sign_body(x_hbm, idxcat, h1_hbm, h2_hbm, h3_hbm, y0_hbm, y1_hbm, y2_hbm,
               dacc, acc, zb, zb2, ones, idxv, gbuf0, gbuf1, dvb, disv,
               sbuf, hb, yb, sem_g0, sem_g1, sem_i, sem_s0, sem_s1):
    s = lax.axis_index("s")
    zv = jnp.zeros((L,), _F32)
    ov = jnp.ones((L,), _F32)
    base = s * NROW

    def fz(i, _):
        zb[pl.ds(i * L, L)] = zv
        return 0
    lax.fori_loop(0, 640 // L, fz, 0)

    def fz2(r, _):
        for k in range(D // L):
            zb2[r, pl.ds(k * L, L)] = zv
        return 0
    lax.fori_loop(0, 40, fz2, 0)

    def fo(i, _):
        ones[pl.ds(i * L, L)] = ov
        return 0
    lax.fori_loop(0, CHUNK // L, fo, 0)

    def zacc(ncopy):
        def zc(j, _):
            pltpu.sync_copy(zb2, acc.at[pl.ds(base + j * 40, 40)])
            return 0
        lax.fori_loop(0, ncopy, zc, 0)

    @pl.when(s < 15)
    def _():
        pltpu.sync_copy(zb, dacc.at[pl.ds(base, NROW)])
        zacc(NROW // 40)

    @pl.when(s == 15)
    def _():
        pltpu.sync_copy(zb.at[pl.ds(0, NROW15)], dacc.at[pl.ds(base, NROW15)])
        zacc(NROW15 // 40)

    plsc.subcore_barrier()

    pltpu.sync_copy(idxcat.at[s, pl.ds(0, SUP)], idxv.at[0])
    pltpu.async_copy(idxcat.at[s, pl.ds(SUP, SUP)], idxv.at[1], sem_i)

    def deg_loop(g, _):
        p = g % 2
        cur = idxv.at[p]
        for b in range(SUP):
            pltpu.sync_copy(ones, dacc.at[cur.at[b, 1]], add=True)

        @pl.when(g < NSUP - 1)
        def _():
            pltpu.make_async_copy(idxcat.at[s, pl.ds(0, SUP)], idxv.at[1 - p],
                                  sem_i).wait()

        @pl.when(g < NSUP - 2)
        def _():
            off = pl.multiple_of((g + 2) * SUP, 8)
            pltpu.async_copy(idxcat.at[s, pl.ds(off, SUP)], idxv.at[p], sem_i)
        return 0
    with jax.named_scope("ph_deg"):
        lax.fori_loop(0, NSUP, deg_loop, 0)
    plsc.subcore_barrier()

    def prep_tail(nrow):
        pltpu.sync_copy(dacc.at[pl.ds(base, nrow)], dvb.at[pl.ds(0, nrow)])

        def gdis(i, _):
            dv = dvb[pl.ds(i * L, L)]
            disv[pl.ds(i * L, L)] = _rsqrt16(dv)
            return 0
        lax.fori_loop(0, nrow // L, gdis, 0)

        def gy(gi, _):
            r0 = base + gi * 8
            pltpu.sync_copy(x_hbm.at[pl.ds(r0, 8)], sbuf)
            dv8 = disv[pl.ds(gi * 8, L)]
            for j in range(8):
                dj = dv8[j]
                for k in range(D // L):
                    yb[j, pl.ds(k * L, L)] = sbuf[j, pl.ds(k * L, L)] * dj
            pltpu.sync_copy(yb, y0_hbm.at[pl.ds(r0, 8)])
            return 0
        lax.fori_loop(0, nrow // 8, gy, 0)

    with jax.named_scope("ph_prep"):
        @pl.when(s < 15)
        def _():
            prep_tail(NROW)

        @pl.when(s == 15)
        def _():
            prep_tail(NROW15)
    plsc.subcore_barrier()

    for hop in range(3):
        y_src = (y0_hbm, y1_hbm, y2_hbm)[hop]
        h_dst = (h1_hbm, h2_hbm, h3_hbm)[hop]
        y_dst = (y1_hbm, y2_hbm, None)[hop]

        pltpu.sync_copy(idxcat.at[s, pl.ds(0, SUP)], idxv.at[0])
        pltpu.async_copy(idxcat.at[s, pl.ds(SUP, SUP)], idxv.at[1], sem_i)
        pltpu.async_copy(y_src.at[idxv.at[0, 0, 0]], gbuf0, sem_g0)

        def sup_loop(g, _, y_src=y_src):
            p = g % 2
            cur = idxv.at[p]
            for b in range(SUP):
                buf, nbuf = (gbuf0, gbuf1) if b % 2 == 0 else (gbuf1, gbuf0)
                semb, semn = ((sem_g0, sem_g1) if b % 2 == 0 else
                              (sem_g1, sem_g0))
                sscur, ssnxt = ((sem_s0, sem_s1) if b % 2 == 0 else
                                (sem_s1, sem_s0))
                pltpu.make_async_copy(y_src.at[cur.at[b, 0]], buf,
                                      semb).wait()
                if b < SUP - 1:
                    pltpu.async_copy(buf, acc.at[cur.at[b, 1]], sscur,
                                     add=True)
                    if b == 0:
                        @pl.when(g > 0)
                        def _():
                            pltpu.make_async_copy(gbuf1, acc.at[cur.at[b, 1]],
                                                  ssnxt).wait()
                    else:
                        pltpu.make_async_copy(gbuf1, acc.at[cur.at[b, 1]],
                                              ssnxt).wait()
                    pltpu.async_copy(y_src.at[cur.at[b + 1, 0]], nbuf, semn)
                else:
                    @pl.when(g < NSUP - 1)
                    def _():
                        pltpu.async_copy(buf, acc.at[cur.at[b, 1]], sscur,
                                         add=True)
                        nxt = idxv.at[1 - p]
                        pltpu.make_async_copy(idxcat.at[s, pl.ds(0, SUP)],
                                              nxt, sem_i).wait()
                        pltpu.make_async_copy(gbuf1, acc.at[cur.at[b, 1]],
                                              ssnxt).wait()
                        pltpu.async_copy(y_src.at[nxt.at[0, 0]], nbuf, semn)

                    @pl.when(g == NSUP - 1)
                    def _():
                        pltpu.sync_copy(buf, acc.at[cur.at[b, 1]], add=True)

                    @pl.when(g < NSUP - 2)
                    def _():
                        off = pl.multiple_of((g + 2) * SUP, 8)
                        pltpu.async_copy(idxcat.at[s, pl.ds(off, SUP)],
                                         idxv.at[p], sem_i)
            return 0
        with jax.named_scope("ph_edge%d" % hop):
            lax.fori_loop(0, NSUP, sup_loop, 0)
            pltpu.make_async_copy(gbuf0, acc.at[idxv.at[0, 0, 1]],
                                  sem_s0).wait()
        plsc.subcore_barrier()

        def node_tail(nrow, hop=hop, h_dst=h_dst, y_dst=y_dst):
            def gn(gi, _):
                r0 = base + gi * 8
                pltpu.sync_copy(acc.at[pl.ds(r0, 8)], sbuf)
                dv8 = disv[pl.ds(gi * 8, L)]
                for j in range(8):
                    dj = dv8[j]
                    dj2 = dj * dj
                    for k in range(D // L):
                        v = sbuf[j, pl.ds(k * L, L)]
                        hb[j, pl.ds(k * L, L)] = v * dj
                        if y_dst is not None:
                            yb[j, pl.ds(k * L, L)] = v * dj2
                pltpu.sync_copy(hb, h_dst.at[pl.ds(r0, 8)])
                if y_dst is not None:
                    pltpu.sync_copy(yb, y_dst.at[pl.ds(r0, 8)])
                return 0
            lax.fori_loop(0, nrow // 8, gn, 0)
            if hop < 2:
                def zc2(j, _):
                    pltpu.sync_copy(zb2, acc.at[pl.ds(base + j * 40, 40)])
                    return 0
                lax.fori_loop(0, nrow // 40, zc2, 0)

        with jax.named_scope("ph_node%d" % hop):
            @pl.when(s < 15)
            def _():
                node_tail(NROW)

            @pl.when(s == 15)
            def _():
                node_tail(NROW15)

        if hop < 2:
            plsc.subcore_barrier()


_sign = functools.partial(
    pl.kernel,
    out_type=tuple(jax.ShapeDtypeStruct((N, D), _F32) for _ in range(6)),
    mesh=_MESH,
    interpret=_INTERP,
    scratch_types=[
        pltpu.VMEM_SHARED((N + 8,), _F32),
        pltpu.VMEM_SHARED((N + 8, D), _F32),
        pltpu.VMEM((640,), _F32),
        pltpu.VMEM((40, D), _F32),
        pltpu.VMEM((CHUNK,), _F32),
        pltpu.VMEM((2, SUP, 2, CHUNK), jnp.int32),
        pltpu.VMEM((CHUNK, D), _F32),
        pltpu.VMEM((CHUNK, D), _F32),
        pltpu.VMEM((640,), _F32),
        pltpu.VMEM((NROW + L,), _F32),
        pltpu.VMEM((8, D), _F32),
        pltpu.VMEM((8, D), _F32),
        pltpu.VMEM((8, D), _F32),
        pltpu.SemaphoreType.DMA,
        pltpu.SemaphoreType.DMA,
        pltpu.SemaphoreType.DMA,
        pltpu.SemaphoreType.DMA,
        pltpu.SemaphoreType.DMA,
    ],
)(_sign_body)


def _mlp_body(xr, h1r, h2r, h3r, w1r, b1r, w2r, b2r, w3r, b3r, outr):
    t = jnp.dot(xr[...], w1r[0:D, :], preferred_element_type=_F32)
    t += jnp.dot(h1r[...], w1r[D:2 * D, :], preferred_element_type=_F32)
    t += jnp.dot(h2r[...], w1r[2 * D:3 * D, :], preferred_element_type=_F32)
    t += jnp.dot(h3r[...], w1r[3 * D:4 * D, :], preferred_element_type=_F32)
    t = jnp.maximum(t + b1r[...], 0.0)
    t = jnp.maximum(jnp.dot(t, w2r[...], preferred_element_type=_F32)
                    + b2r[...], 0.0)
    outr[...] = jnp.dot(t, w3r[...], preferred_element_type=_F32) + b3r[...]


def _mlp(x, h1, h2, h3, W1, b1, W2, b2, W3, b3):
    R = 1000
    full = lambda i: (0, 0)
    blk = lambda i: (i, 0)
    return pl.pallas_call(
        _mlp_body,
        grid=(N // R,),
        in_specs=[pl.BlockSpec((R, D), blk)] * 4 + [
            pl.BlockSpec((4 * D, HIDDEN), full),
            pl.BlockSpec((1, HIDDEN), full),
            pl.BlockSpec((HIDDEN, HIDDEN), full),
            pl.BlockSpec((1, HIDDEN), full),
            pl.BlockSpec((HIDDEN, OUT_C), full),
            pl.BlockSpec((1, OUT_C), full),
        ],
        out_specs=pl.BlockSpec((R, OUT_C), blk),
        out_shape=jax.ShapeDtypeStruct((N, OUT_C), _F32),
        interpret=_INTERP,
    )(x, h1, h2, h3, W1, b1.reshape(1, HIDDEN), W2, b2.reshape(1, HIDDEN),
      W3, b3.reshape(1, OUT_C))


def kernel(x, edge_index, W1, b1, W2, b2, W3, b3):
    pad = EPAD - E
    row = jnp.concatenate(
        [edge_index[0], jnp.full((pad,), N, jnp.int32)]).reshape(
            NS, CPT, CHUNK)
    col = jnp.concatenate(
        [edge_index[1], jnp.zeros((pad,), jnp.int32)]).reshape(
            NS, CPT, CHUNK)
    idxcat = jnp.stack([col, row], axis=2)
    h1, h2, h3, _, _, _ = _sign(x, idxcat)
    return _mlp(x, h1, h2, h3, W1, b1, W2, b2, W3, b3)


# speedup vs baseline: 2.5603x; 2.5603x over previous
import functools

import jax
import jax.numpy as jnp
from jax import lax
from jax.experimental import pallas as pl
from jax.experimental.pallas import tpu as pltpu
from jax.experimental.pallas import tpu_sc as plsc

N = 10000
E = 320000
D = 128
HIDDEN = 256
OUT_C = 128

L = 16
NC = 2
NS = 16
CHUNK = 100
NCHUNK = E // CHUNK
CPT = NCHUNK // NS
SUP = 8
NSUP = CPT // SUP
NROW = 640
NROW15 = N - 15 * NROW

_MESH = plsc.VectorSubcoreMesh(core_axis_name="c", subcore_axis_name="s",
                               num_cores=NC, num_subcores=NS)
_F32 = jnp.float32


def _rsqrt16(d):
    i = lax.bitcast_convert_type(d, jnp.int32)
    i = jnp.int32(0x5F3759DF) - lax.shift_right_logical(i, 1)
    y = lax.bitcast_convert_type(i, _F32)
    half = 0.5 * d
    for _ in range(3):
        y = y * (1.5 - half * y * y)
    return jnp.where(d > 0.5, y, 0.0)


def _sign_body(x_hbm, idxcat, h1_hbm, h2_hbm, h3_hbm, y0_hbm, y1_hbm, y2_hbm,
               dacc, acc, zb, zb2, ones, idxv, gbuf0, gbuf1, dvb, disv,
               sbuf, hb, yb, sem_g0, sem_g1, sem_i):
    s = lax.axis_index("s")
    zv = jnp.zeros((L,), _F32)
    ov = jnp.ones((L,), _F32)
    base = s * NROW

    def fz(i, _):
        zb[pl.ds(i * L, L)] = zv
        return 0
    lax.fori_loop(0, 640 // L, fz, 0)

    def fz2(r, _):
        for k in range(D // L):
            zb2[r, pl.ds(k * L, L)] = zv
        return 0
    lax.fori_loop(0, 40, fz2, 0)

    def fo(i, _):
        ones[pl.ds(i * L, L)] = ov
        return 0
    lax.fori_loop(0, 112 // L, fo, 0)

    def zacc(ncopy):
        def zc(j, _):
            pltpu.sync_copy(zb2, acc.at[pl.ds(base + j * 40, 40)])
            return 0
        lax.fori_loop(0, ncopy, zc, 0)

    @pl.when(s < 15)
    def _():
        pltpu.sync_copy(zb, dacc.at[pl.ds(base, NROW)])
        zacc(NROW // 40)

    @pl.when(s == 15)
    def _():
        pltpu.sync_copy(zb.at[pl.ds(0, NROW15)], dacc.at[pl.ds(base, NROW15)])
        zacc(NROW15 // 40)

    plsc.subcore_barrier()

    pltpu.sync_copy(idxcat.at[s, pl.ds(0, SUP)], idxv.at[0])
    pltpu.async_copy(idxcat.at[s, pl.ds(SUP, SUP)], idxv.at[1], sem_i)

    def deg_loop(g, _):
        p = g % 2
        cur = idxv.at[p]
        for b in range(SUP):
            pltpu.sync_copy(ones.at[pl.ds(0, CHUNK)], dacc.at[cur.at[b, 1]],
                            add=True)

        @pl.when(g < NSUP - 1)
        def _():
            pltpu.make_async_copy(idxcat.at[s, pl.ds(0, SUP)], idxv.at[1 - p],
                                  sem_i).wait()

        @pl.when(g < NSUP - 2)
        def _():
            off = pl.multiple_of((g + 2) * SUP, 8)
            pltpu.async_copy(idxcat.at[s, pl.ds(off, SUP)], idxv.at[p], sem_i)
        return 0
    with jax.named_scope("ph_deg"):
        lax.fori_loop(0, NSUP, deg_loop, 0)
    plsc.subcore_barrier()

    def prep_tail(nrow):
        pltpu.sync_copy(dacc.at[pl.ds(base, nrow)], dvb.at[pl.ds(0, nrow)])

        def gdis(i, _):
            dv = dvb[pl.ds(i * L, L)]
            disv[pl.ds(i * L, L)] = _rsqrt16(dv)
            return 0
        lax.fori_loop(0, nrow // L, gdis, 0)

        def gy(gi, _):
            r0 = base + gi * 8
            pltpu.sync_copy(x_hbm.at[pl.ds(r0, 8)], sbuf)
            dv8 = disv[pl.ds(gi * 8, L)]
            for j in range(8):
                dj = dv8[j]
                for k in range(D // L):
                    yb[j, pl.ds(k * L, L)] = sbuf[j, pl.ds(k * L, L)] * dj
            pltpu.sync_copy(yb, y0_hbm.at[pl.ds(r0, 8)])
            return 0
        lax.fori_loop(0, nrow // 8, gy, 0)

    with jax.named_scope("ph_prep"):
        @pl.when(s < 15)
        def _():
            prep_tail(NROW)

        @pl.when(s == 15)
        def _():
            prep_tail(NROW15)
    plsc.subcore_barrier()

    for hop in range(3):
        y_src = (y0_hbm, y1_hbm, y2_hbm)[hop]
        h_dst = (h1_hbm, h2_hbm, h3_hbm)[hop]
        y_dst = (y1_hbm, y2_hbm, None)[hop]

        pltpu.sync_copy(idxcat.at[s, pl.ds(0, SUP)], idxv.at[0])
        pltpu.async_copy(idxcat.at[s, pl.ds(SUP, SUP)], idxv.at[1], sem_i)
        pltpu.async_copy(y_src.at[idxv.at[0, 0, 0]], gbuf0, sem_g0)

        def sup_loop(g, _, y_src=y_src):
            p = g % 2
            cur = idxv.at[p]
            for b in range(SUP):
                buf, nbuf = (gbuf0, gbuf1) if b % 2 == 0 else (gbuf1, gbuf0)
                semb, semn = ((sem_g0, sem_g1) if b % 2 == 0 else
                              (sem_g1, sem_g0))
                pltpu.make_async_copy(y_src.at[cur.at[b, 0]], buf,
                                      semb).wait()
                if b < SUP - 1:
                    pltpu.async_copy(y_src.at[cur.at[b + 1, 0]], nbuf, semn)
                    pltpu.sync_copy(buf, acc.at[cur.at[b, 1]], add=True)
                else:
                    @pl.when(g < NSUP - 1)
                    def _():
                        nxt = idxv.at[1 - p]
                        pltpu.make_async_copy(idxcat.at[s, pl.ds(0, SUP)],
                                              nxt, sem_i).wait()
                        pltpu.async_copy(y_src.at[nxt.at[0, 0]], nbuf, semn)
                    pltpu.sync_copy(buf, acc.at[cur.at[b, 1]], add=True)

                    @pl.when(g < NSUP - 2)
                    def _():
                        off = pl.multiple_of((g + 2) * SUP, 8)
                        pltpu.async_copy(idxcat.at[s, pl.ds(off, SUP)],
                                         idxv.at[p], sem_i)
            return 0
        with jax.named_scope("ph_edge%d" % hop):
            lax.fori_loop(0, NSUP, sup_loop, 0)
        plsc.subcore_barrier()

        def node_tail(nrow, hop=hop, h_dst=h_dst, y_dst=y_dst):
            def gn(gi, _):
                r0 = base + gi * 8
                pltpu.sync_copy(acc.at[pl.ds(r0, 8)], sbuf)
                dv8 = disv[pl.ds(gi * 8, L)]
                for j in range(8):
                    dj = dv8[j]
                    dj2 = dj * dj
                    for k in range(D // L):
                        v = sbuf[j, pl.ds(k * L, L)]
                        hb[j, pl.ds(k * L, L)] = v * dj
                        if y_dst is not None:
                            yb[j, pl.ds(k * L, L)] = v * dj2
                pltpu.sync_copy(hb, h_dst.at[pl.ds(r0, 8)])
                if y_dst is not None:
                    pltpu.sync_copy(yb, y_dst.at[pl.ds(r0, 8)])
                return 0
            lax.fori_loop(0, nrow // 8, gn, 0)
            if hop < 2:
                def zc2(j, _):
                    pltpu.sync_copy(zb2, acc.at[pl.ds(base + j * 40, 40)])
                    return 0
                lax.fori_loop(0, nrow // 40, zc2, 0)

        with jax.named_scope("ph_node%d" % hop):
            @pl.when(s < 15)
            def _():
                node_tail(NROW)

            @pl.when(s == 15)
            def _():
                node_tail(NROW15)

        if hop < 2:
            plsc.subcore_barrier()


_sign = functools.partial(
    pl.kernel,
    out_type=tuple(jax.ShapeDtypeStruct((N, D), _F32) for _ in range(6)),
    mesh=_MESH,
    scratch_types=[
        pltpu.VMEM_SHARED((N,), _F32),
        pltpu.VMEM_SHARED((N, D), _F32),
        pltpu.VMEM((640,), _F32),
        pltpu.VMEM((40, D), _F32),
        pltpu.VMEM((112,), _F32),
        pltpu.VMEM((2, SUP, 2, CHUNK), jnp.int32),
        pltpu.VMEM((CHUNK, D), _F32),
        pltpu.VMEM((CHUNK, D), _F32),
        pltpu.VMEM((640,), _F32),
        pltpu.VMEM((NROW + L,), _F32),
        pltpu.VMEM((8, D), _F32),
        pltpu.VMEM((8, D), _F32),
        pltpu.VMEM((8, D), _F32),
        pltpu.SemaphoreType.DMA,
        pltpu.SemaphoreType.DMA,
        pltpu.SemaphoreType.DMA,
    ],
)(_sign_body)


def _mlp_body(xr, h1r, h2r, h3r, w1r, b1r, w2r, b2r, w3r, b3r, outr):
    t = jnp.dot(xr[...], w1r[0:D, :], preferred_element_type=_F32)
    t += jnp.dot(h1r[...], w1r[D:2 * D, :], preferred_element_type=_F32)
    t += jnp.dot(h2r[...], w1r[2 * D:3 * D, :], preferred_element_type=_F32)
    t += jnp.dot(h3r[...], w1r[3 * D:4 * D, :], preferred_element_type=_F32)
    t = jnp.maximum(t + b1r[...], 0.0)
    t = jnp.maximum(jnp.dot(t, w2r[...], preferred_element_type=_F32)
                    + b2r[...], 0.0)
    outr[...] = jnp.dot(t, w3r[...], preferred_element_type=_F32) + b3r[...]


def _mlp(x, h1, h2, h3, W1, b1, W2, b2, W3, b3):
    R = 1000
    full = lambda i: (0, 0)
    blk = lambda i: (i, 0)
    return pl.pallas_call(
        _mlp_body,
        grid=(N // R,),
        in_specs=[pl.BlockSpec((R, D), blk)] * 4 + [
            pl.BlockSpec((4 * D, HIDDEN), full),
            pl.BlockSpec((1, HIDDEN), full),
            pl.BlockSpec((HIDDEN, HIDDEN), full),
            pl.BlockSpec((1, HIDDEN), full),
            pl.BlockSpec((HIDDEN, OUT_C), full),
            pl.BlockSpec((1, OUT_C), full),
        ],
        out_specs=pl.BlockSpec((R, OUT_C), blk),
        out_shape=jax.ShapeDtypeStruct((N, OUT_C), _F32),
        )(x, h1, h2, h3, W1, b1.reshape(1, HIDDEN), W2, b2.reshape(1, HIDDEN),
      W3, b3.reshape(1, OUT_C))


def kernel(x, edge_index, W1, b1, W2, b2, W3, b3):
    row = edge_index[0].reshape(NS, CPT, CHUNK)
    col = edge_index[1].reshape(NS, CPT, CHUNK)
    idxcat = jnp.stack([col, row], axis=2)
    h1, h2, h3, _, _, _ = _sign(x, idxcat)
    return _mlp(x, h1, h2, h3, W1, b1, W2, b2, W3, b3)


# speedup vs baseline: 2.6892x; 1.0504x over previous
import functools

import jax
import jax.numpy as jnp
from jax import lax
from jax.experimental import pallas as pl
from jax.experimental.pallas import tpu as pltpu
from jax.experimental.pallas import tpu_sc as plsc

N = 10000
E = 320000
D = 128
HIDDEN = 256
OUT_C = 128

L = 16
NC = 2
NS = 16
CHUNK = 100
NCHUNK = E // CHUNK
CPT = NCHUNK // NS
SUP = 8
NSUP = CPT // SUP
NROW = 640
NROW15 = N - 15 * NROW

_MESH = plsc.VectorSubcoreMesh(core_axis_name="c", subcore_axis_name="s",
                               num_cores=NC, num_subcores=NS)
_F32 = jnp.float32


def _rsqrt16(d):
    i = lax.bitcast_convert_type(d, jnp.int32)
    i = jnp.int32(0x5F3759DF) - lax.shift_right_logical(i, 1)
    y = lax.bitcast_convert_type(i, _F32)
    half = 0.5 * d
    for _ in range(3):
        y = y * (1.5 - half * y * y)
    return jnp.where(d > 0.5, y, 0.0)


def _sign_body(x_hbm, idxcat, h1_hbm, h2_hbm, h3_hbm, y0_hbm, y1_hbm, y2_hbm,
               dacc, acc, zb, zb2, ones, idxv, gbuf0, gbuf1, dvb, disv,
               sbuf, hb, yb, sem_g0, sem_g1, sem_i):
    s = lax.axis_index("s")
    zv = jnp.zeros((L,), _F32)
    ov = jnp.ones((L,), _F32)
    base = s * NROW

    def fz(i, _):
        zb[pl.ds(i * L, L)] = zv
        return 0
    lax.fori_loop(0, 640 // L, fz, 0)

    def fz2(r, _):
        for k in range(D // L):
            zb2[r, pl.ds(k * L, L)] = zv
        return 0
    lax.fori_loop(0, 40, fz2, 0)

    def fo(i, _):
        ones[pl.ds(i * L, L)] = ov
        return 0
    lax.fori_loop(0, 112 // L, fo, 0)

    def zacc(ncopy):
        def zc(j, _):
            pltpu.sync_copy(zb2, acc.at[pl.ds(base + j * 40, 40)])
            return 0
        lax.fori_loop(0, ncopy, zc, 0)

    @pl.when(s < 15)
    def _():
        pltpu.sync_copy(zb, dacc.at[pl.ds(base, NROW)])
        zacc(NROW // 40)

    @pl.when(s == 15)
    def _():
        pltpu.sync_copy(zb.at[pl.ds(0, NROW15)], dacc.at[pl.ds(base, NROW15)])
        zacc(NROW15 // 40)

    plsc.subcore_barrier()

    pltpu.sync_copy(idxcat.at[s, pl.ds(0, SUP)], idxv.at[0])
    pltpu.async_copy(idxcat.at[s, pl.ds(SUP, SUP)], idxv.at[1], sem_i)

    def deg_loop(g, _):
        p = g % 2
        cur = idxv.at[p]
        for b in range(SUP):
            pltpu.sync_copy(ones.at[pl.ds(0, CHUNK)], dacc.at[cur.at[b, 1]],
                            add=True)

        @pl.when(g < NSUP - 1)
        def _():
            pltpu.make_async_copy(idxcat.at[s, pl.ds(0, SUP)], idxv.at[1 - p],
                                  sem_i).wait()

        @pl.when(g < NSUP - 2)
        def _():
            off = pl.multiple_of((g + 2) * SUP, 8)
            pltpu.async_copy(idxcat.at[s, pl.ds(off, SUP)], idxv.at[p], sem_i)
        return 0
    with jax.named_scope("ph_deg"):
        lax.fori_loop(0, NSUP, deg_loop, 0)
    plsc.subcore_barrier()

    def prep_tail(nrow):
        pltpu.sync_copy(dacc.at[pl.ds(base, nrow)], dvb.at[pl.ds(0, nrow)])

        def gdis(i, _):
            dv = dvb[pl.ds(i * L, L)]
            disv[pl.ds(i * L, L)] = _rsqrt16(dv)
            return 0
        lax.fori_loop(0, nrow // L, gdis, 0)

        def gy(gi, _):
            r0 = base + gi * 16
            pltpu.sync_copy(x_hbm.at[pl.ds(r0, 16)], sbuf)
            dv16 = disv[pl.ds(gi * 16, L)]
            for j in range(16):
                dj = dv16[j]
                for k in range(D // L):
                    yb[j, pl.ds(k * L, L)] = sbuf[j, pl.ds(k * L, L)] * dj
            pltpu.sync_copy(yb, y0_hbm.at[pl.ds(r0, 16)])
            return 0
        lax.fori_loop(0, nrow // 16, gy, 0)

    with jax.named_scope("ph_prep"):
        @pl.when(s < 15)
        def _():
            prep_tail(NROW)

        @pl.when(s == 15)
        def _():
            prep_tail(NROW15)
    plsc.subcore_barrier()

    for hop in range(3):
        y_src = (y0_hbm, y1_hbm, y2_hbm)[hop]
        h_dst = (h1_hbm, h2_hbm, h3_hbm)[hop]
        y_dst = (y1_hbm, y2_hbm, None)[hop]

        pltpu.sync_copy(idxcat.at[s, pl.ds(0, SUP)], idxv.at[0])
        pltpu.async_copy(idxcat.at[s, pl.ds(SUP, SUP)], idxv.at[1], sem_i)
        pltpu.async_copy(y_src.at[idxv.at[0, 0, 0]], gbuf0, sem_g0)

        def sup_loop(g, _, y_src=y_src):
            p = g % 2
            cur = idxv.at[p]
            for b in range(SUP):
                buf, nbuf = (gbuf0, gbuf1) if b % 2 == 0 else (gbuf1, gbuf0)
                semb, semn = ((sem_g0, sem_g1) if b % 2 == 0 else
                              (sem_g1, sem_g0))
                pltpu.make_async_copy(y_src.at[cur.at[b, 0]], buf,
                                      semb).wait()
                if b < SUP - 1:
                    pltpu.async_copy(y_src.at[cur.at[b + 1, 0]], nbuf, semn)
                    pltpu.sync_copy(buf, acc.at[cur.at[b, 1]], add=True)
                else:
                    @pl.when(g < NSUP - 1)
                    def _():
                        nxt = idxv.at[1 - p]
                        pltpu.make_async_copy(idxcat.at[s, pl.ds(0, SUP)],
                                              nxt, sem_i).wait()
                        pltpu.async_copy(y_src.at[nxt.at[0, 0]], nbuf, semn)
                    pltpu.sync_copy(buf, acc.at[cur.at[b, 1]], add=True)

                    @pl.when(g < NSUP - 2)
                    def _():
                        off = pl.multiple_of((g + 2) * SUP, 8)
                        pltpu.async_copy(idxcat.at[s, pl.ds(off, SUP)],
                                         idxv.at[p], sem_i)
            return 0
        with jax.named_scope("ph_edge%d" % hop):
            lax.fori_loop(0, NSUP, sup_loop, 0)
        plsc.subcore_barrier()

        def node_tail(nrow, hop=hop, h_dst=h_dst, y_dst=y_dst):
            def gn(gi, _):
                r0 = base + gi * 16
                pltpu.sync_copy(acc.at[pl.ds(r0, 16)], sbuf)
                dv16 = disv[pl.ds(gi * 16, L)]
                for j in range(16):
                    dj = dv16[j]
                    dj2 = dj * dj
                    for k in range(D // L):
                        v = sbuf[j, pl.ds(k * L, L)]
                        hb[j, pl.ds(k * L, L)] = v * dj
                        if y_dst is not None:
                            yb[j, pl.ds(k * L, L)] = v * dj2
                pltpu.sync_copy(hb, h_dst.at[pl.ds(r0, 16)])
                if y_dst is not None:
                    pltpu.sync_copy(yb, y_dst.at[pl.ds(r0, 16)])
                return 0
            lax.fori_loop(0, nrow // 16, gn, 0)
            if hop < 2:
                def zc2(j, _):
                    pltpu.sync_copy(zb2, acc.at[pl.ds(base + j * 40, 40)])
                    return 0
                lax.fori_loop(0, nrow // 40, zc2, 0)

        with jax.named_scope("ph_node%d" % hop):
            @pl.when(s < 15)
            def _():
                node_tail(NROW)

            @pl.when(s == 15)
            def _():
                node_tail(NROW15)

        if hop < 2:
            plsc.subcore_barrier()


_sign = functools.partial(
    pl.kernel,
    out_type=tuple(jax.ShapeDtypeStruct((N, D), _F32) for _ in range(6)),
    mesh=_MESH,
    scratch_types=[
        pltpu.VMEM_SHARED((N,), _F32),
        pltpu.VMEM_SHARED((N, D), _F32),
        pltpu.VMEM((640,), _F32),
        pltpu.VMEM((40, D), _F32),
        pltpu.VMEM((112,), _F32),
        pltpu.VMEM((2, SUP, 2, CHUNK), jnp.int32),
        pltpu.VMEM((CHUNK, D), _F32),
        pltpu.VMEM((CHUNK, D), _F32),
        pltpu.VMEM((640,), _F32),
        pltpu.VMEM((NROW + L,), _F32),
        pltpu.VMEM((16, D), _F32),
        pltpu.VMEM((16, D), _F32),
        pltpu.VMEM((16, D), _F32),
        pltpu.SemaphoreType.DMA,
        pltpu.SemaphoreType.DMA,
        pltpu.SemaphoreType.DMA,
    ],
)(_sign_body)


def _mlp_body(xr, h1r, h2r, h3r, w1r, b1r, w2r, b2r, w3r, b3r, outr):
    t = jnp.dot(xr[...], w1r[0:D, :], preferred_element_type=_F32)
    t += jnp.dot(h1r[...], w1r[D:2 * D, :], preferred_element_type=_F32)
    t += jnp.dot(h2r[...], w1r[2 * D:3 * D, :], preferred_element_type=_F32)
    t += jnp.dot(h3r[...], w1r[3 * D:4 * D, :], preferred_element_type=_F32)
    t = jnp.maximum(t + b1r[...], 0.0)
    t = jnp.maximum(jnp.dot(t, w2r[...], preferred_element_type=_F32)
                    + b2r[...], 0.0)
    outr[...] = jnp.dot(t, w3r[...], preferred_element_type=_F32) + b3r[...]


def _mlp(x, h1, h2, h3, W1, b1, W2, b2, W3, b3):
    R = 1000
    full = lambda i: (0, 0)
    blk = lambda i: (i, 0)
    return pl.pallas_call(
        _mlp_body,
        grid=(N // R,),
        in_specs=[pl.BlockSpec((R, D), blk)] * 4 + [
            pl.BlockSpec((4 * D, HIDDEN), full),
            pl.BlockSpec((1, HIDDEN), full),
            pl.BlockSpec((HIDDEN, HIDDEN), full),
            pl.BlockSpec((1, HIDDEN), full),
            pl.BlockSpec((HIDDEN, OUT_C), full),
            pl.BlockSpec((1, OUT_C), full),
        ],
        out_specs=pl.BlockSpec((R, OUT_C), blk),
        out_shape=jax.ShapeDtypeStruct((N, OUT_C), _F32),
        )(x, h1, h2, h3, W1, b1.reshape(1, HIDDEN), W2, b2.reshape(1, HIDDEN),
      W3, b3.reshape(1, OUT_C))


def kernel(x, edge_index, W1, b1, W2, b2, W3, b3):
    row = edge_index[0].reshape(NS, CPT, CHUNK)
    col = edge_index[1].reshape(NS, CPT, CHUNK)
    idxcat = jnp.stack([col, row], axis=2)
    h1, h2, h3, _, _, _ = _sign(x, idxcat)
    return _mlp(x, h1, h2, h3, W1, b1, W2, b2, W3, b3)
